# Initial kernel scaffold; baseline (speedup 1.0000x reference)
import functools

import jax
import jax.numpy as jnp
from jax import lax
from jax.experimental import pallas as pl
from jax.experimental.pallas import tpu as pltpu
from jax.experimental.pallas import tpu_sc as plsc

B, N, C, H = 4096, 9, 96, 3
DH = C // H
E = 16384

NC, NS = 2, 16
NW = NC * NS
R_COLS = 256
A_COLS = H * N * N

_BB = 256
_EB = 256


def _tc_qkv_body(x_ref, g1_ref, b1_ref, wt_ref, q_ref, k_ref, v_ref):
    g = g1_ref[...]
    b = b1_ref[...]
    wt = wt_ref[...]
    for n in range(N):
        xn = x_ref[:, n, :]
        mu = jnp.mean(xn, axis=-1, keepdims=True)
        var = jnp.mean((xn - mu) ** 2, axis=-1, keepdims=True)
        x1 = (xn - mu) * jax.lax.rsqrt(var + 1e-5) * g + b
        qkv = jnp.dot(x1, wt, preferred_element_type=jnp.float32)
        for h in range(H):
            q_ref[:, h * N + n, :] = qkv[:, h * DH:(h + 1) * DH]
            k_ref[:, h * N + n, :] = qkv[:, C + h * DH:C + (h + 1) * DH]
        v_ref[:, n, :] = qkv[:, 2 * C:3 * C]


def _tc_qkv(x, g1, b1, wqkv_t):
    grid = (B // _BB,)
    return pl.pallas_call(
        _tc_qkv_body,
        grid=grid,
        in_specs=[
            pl.BlockSpec((_BB, N, C), lambda i: (i, 0, 0)),
            pl.BlockSpec((1, C), lambda i: (0, 0)),
            pl.BlockSpec((1, C), lambda i: (0, 0)),
            pl.BlockSpec((C, 3 * C), lambda i: (0, 0)),
        ],
        out_specs=[
            pl.BlockSpec((_BB, H * N, DH), lambda i: (i, 0, 0)),
            pl.BlockSpec((_BB, H * N, DH), lambda i: (i, 0, 0)),
            pl.BlockSpec((_BB, N, C), lambda i: (i, 0, 0)),
        ],
        out_shape=[
            jax.ShapeDtypeStruct((B, H * N, DH), jnp.float32),
            jax.ShapeDtypeStruct((B, H * N, DH), jnp.float32),
            jax.ShapeDtypeStruct((B, N, C), jnp.float32),
        ],
    )(x, g1, b1, wqkv_t)


_G_CH = 64
_EPW = E // NW


def _sc_gather_body(q_hbm, k_hbm, dst_hbm, src_hbm, qg_hbm, kg_hbm,
                    idx_v, buf, sem):
    wid = lax.axis_index("s") * NC + lax.axis_index("c")
    base = wid * _EPW
    for tab, idx_hbm, out in ((q_hbm, dst_hbm, qg_hbm),
                              (k_hbm, src_hbm, kg_hbm)):
        for c in range(_EPW // _G_CH):
            start = base + c * _G_CH
            pltpu.sync_copy(idx_hbm.at[pl.ds(start, _G_CH)], idx_v)
            pltpu.async_copy(tab.at[idx_v], buf, sem).wait()
            pltpu.sync_copy(buf, out.at[pl.ds(start, _G_CH)])


def _sc_gather(q2, k2, dst, src):
    mesh = plsc.VectorSubcoreMesh(core_axis_name="c", subcore_axis_name="s")
    fn = functools.partial(
        pl.kernel,
        mesh=mesh,
        out_type=[
            jax.ShapeDtypeStruct((E, N * C), jnp.float32),
            jax.ShapeDtypeStruct((E, N * C), jnp.float32),
        ],
        scratch_types=[
            pltpu.VMEM((_G_CH,), jnp.int32),
            pltpu.VMEM((_G_CH, N * C), jnp.float32),
            pltpu.SemaphoreType.DMA,
        ],
    )(_sc_gather_body)
    return fn(q2, k2, dst, src)


def _tc_attn_body(q_ref, k_ref, r_ref):
    scale = DH ** -0.5
    parts = []
    for h in range(H):
        qh = q_ref[:, h * N:(h + 1) * N, :]
        kh = k_ref[:, h * N:(h + 1) * N, :]
        lg = jax.lax.dot_general(
            qh, kh, (((2,), (2,)), ((0,), (0,))),
            preferred_element_type=jnp.float32) * scale
        mx = jnp.max(lg, axis=-1, keepdims=True)
        ex = jnp.exp(lg - mx)
        sm = ex / jnp.sum(ex, axis=-1, keepdims=True)
        parts.append(sm.reshape(_EB, N * N))
    ones = jnp.ones((_EB, 1), jnp.float32)
    pad = jnp.zeros((_EB, R_COLS - A_COLS - 1), jnp.float32)
    r_ref[...] = jnp.concatenate(parts + [ones, pad], axis=-1)


def _tc_attn(qg3, kg3):
    grid = (E // _EB,)
    return pl.pallas_call(
        _tc_attn_body,
        grid=grid,
        in_specs=[
            pl.BlockSpec((_EB, H * N, DH), lambda i: (i, 0, 0)),
            pl.BlockSpec((_EB, H * N, DH), lambda i: (i, 0, 0)),
        ],
        out_specs=pl.BlockSpec((_EB, R_COLS), lambda i: (i, 0)),
        out_shape=jax.ShapeDtypeStruct((E, R_COLS), jnp.float32),
    )(qg3, kg3)


_S_CH = 128
_Z_ROWS = B // NS


def _sc_scatter_body(r_hbm, dst_hbm, s2_hbm, idx_v, buf, zbuf, s_sh, sem):
    del sem
    cid = lax.axis_index("c")
    sid = lax.axis_index("s")
    zrow = jnp.zeros((16,), jnp.float32)

    def _zero_row(i, carry):
        for j in range(R_COLS // 16):
            zbuf[i, pl.ds(j * 16, 16)] = zrow
        return carry

    lax.fori_loop(0, _Z_ROWS, _zero_row, 0)
    pltpu.sync_copy(zbuf, s_sh.at[pl.ds(sid * _Z_ROWS, _Z_ROWS)])
    plsc.subcore_barrier()

    base = (cid * NS + sid) * _EPW
    for c in range(_EPW // _S_CH):
        start = base + c * _S_CH
        pltpu.sync_copy(dst_hbm.at[pl.ds(start, _S_CH)], idx_v)
        pltpu.sync_copy(r_hbm.at[pl.ds(start, _S_CH)], buf)
        pltpu.sync_copy(buf, s_sh.at[idx_v], add=True)
    plsc.subcore_barrier()

    pltpu.sync_copy(s_sh.at[pl.ds(sid * _Z_ROWS, _Z_ROWS)], zbuf)
    pltpu.sync_copy(zbuf, s2_hbm.at[cid, pl.ds(sid * _Z_ROWS, _Z_ROWS)])


def _sc_scatter(r, dst):
    mesh = plsc.VectorSubcoreMesh(core_axis_name="c", subcore_axis_name="s")
    fn = functools.partial(
        pl.kernel,
        mesh=mesh,
        out_type=jax.ShapeDtypeStruct((NC, B, R_COLS), jnp.float32),
        scratch_types=[
            pltpu.VMEM((_S_CH,), jnp.int32),
            pltpu.VMEM((_S_CH, R_COLS), jnp.float32),
            pltpu.VMEM((_Z_ROWS, R_COLS), jnp.float32),
            pltpu.VMEM_SHARED((B, R_COLS), jnp.float32),
            pltpu.SemaphoreType.DMA,
        ],
    )(_sc_scatter_body)
    return fn(r, dst)


def _tc_out_body(s2_ref, v_ref, x_ref, g2_ref, b2_ref, wp_ref, bp_ref,
                 wf1_ref, bf1_ref, wf2_ref, bf2_ref, o_ref):
    s = s2_ref[0] + s2_ref[1]
    cnt = s[:, A_COLS:A_COLS + 1]
    recip = 1.0 / jnp.maximum(cnt, 1.0)
    mask = (cnt > 0.0).astype(jnp.float32)
    heads = []
    for h in range(H):
        wh = (s[:, h * N * N:(h + 1) * N * N] * recip).reshape(_BB, N, N)
        vh = v_ref[:, :, h * DH:(h + 1) * DH]
        ah = jax.lax.dot_general(
            wh, vh, (((2,), (1,)), ((0,), (0,))),
            preferred_element_type=jnp.float32)
        heads.append(ah)
    a = jnp.concatenate(heads, axis=-1)

    g2 = g2_ref[...]
    b2 = b2_ref[...]
    wp = wp_ref[...]
    bp = bp_ref[...]
    wf1 = wf1_ref[...]
    bf1 = bf1_ref[...]
    wf2 = wf2_ref[...]
    bf2 = bf2_ref[...]
    inv_sqrt2 = 0.7071067811865476
    for n in range(N):
        yn = (jnp.dot(a[:, n, :], wp, preferred_element_type=jnp.float32)
              + bp) * mask
        x2n = x_ref[:, n, :] + yn
        mu = jnp.mean(x2n, axis=-1, keepdims=True)
        var = jnp.mean((x2n - mu) ** 2, axis=-1, keepdims=True)
        hn = (x2n - mu) * jax.lax.rsqrt(var + 1e-5) * g2 + b2
        f = jnp.dot(hn, wf1, preferred_element_type=jnp.float32) + bf1
        f = 0.5 * f * (1.0 + jax.lax.erf(f * inv_sqrt2))
        on = jnp.dot(f, wf2, preferred_element_type=jnp.float32) + bf2
        o_ref[:, n, :] = x2n + on


def _tc_out(s2, v, x, g2, b2, wproj_t, bproj, wfc1_t, bfc1, wfc2_t, bfc2):
    grid = (B // _BB,)
    return pl.pallas_call(
        _tc_out_body,
        grid=grid,
        in_specs=[
            pl.BlockSpec((NC, _BB, R_COLS), lambda i: (0, i, 0)),
            pl.BlockSpec((_BB, N, C), lambda i: (i, 0, 0)),
            pl.BlockSpec((_BB, N, C), lambda i: (i, 0, 0)),
            pl.BlockSpec((1, C), lambda i: (0, 0)),
            pl.BlockSpec((1, C), lambda i: (0, 0)),
            pl.BlockSpec((C, C), lambda i: (0, 0)),
            pl.BlockSpec((1, C), lambda i: (0, 0)),
            pl.BlockSpec((C, 4 * C), lambda i: (0, 0)),
            pl.BlockSpec((1, 4 * C), lambda i: (0, 0)),
            pl.BlockSpec((4 * C, C), lambda i: (0, 0)),
            pl.BlockSpec((1, C), lambda i: (0, 0)),
        ],
        out_specs=pl.BlockSpec((_BB, N, C), lambda i: (i, 0, 0)),
        out_shape=jax.ShapeDtypeStruct((B, N, C), jnp.float32),
    )(s2, v, x, g2, b2, wproj_t, bproj, wfc1_t, bfc1, wfc2_t, bfc2)


def kernel(x, edge_index, g1, b1, Wqkv, Wproj, bproj, g2, b2,
           Wfc1, bfc1, Wfc2, bfc2):
    src = edge_index[0]
    dst = edge_index[1]

    q, k, v = _tc_qkv(x, g1.reshape(1, C), b1.reshape(1, C), Wqkv.T)

    qg, kg = _sc_gather(q.reshape(B, N * C), k.reshape(B, N * C), dst, src)

    r = _tc_attn(qg.reshape(E, H * N, DH), kg.reshape(E, H * N, DH))

    s2 = _sc_scatter(r, dst)

    out = _tc_out(s2, v, x, g2.reshape(1, C), b2.reshape(1, C),
                  Wproj.T, bproj.reshape(1, C),
                  Wfc1.T, bfc1.reshape(1, 4 * C),
                  Wfc2.T, bfc2.reshape(1, C))
    return out


# speedup vs baseline: 3.4215x; 3.4215x over previous
import functools

import jax
import jax.numpy as jnp
from jax import lax
from jax.experimental import pallas as pl
from jax.experimental.pallas import tpu as pltpu
from jax.experimental.pallas import tpu_sc as plsc

B, N, C, H = 4096, 9, 96, 3
DH = C // H
E = 16384

NC, NS = 2, 16
NW = NC * NS
R_COLS = 256
A_COLS = H * N * N

_BB = 256
GR = H * N + 1
GC = GR * DH
_EB = 256


def _tc_qkv_body(x_ref, g1_ref, b1_ref, wt_ref, q_ref, k_ref, v_ref):
    g = g1_ref[...]
    b = b1_ref[...]
    wt = wt_ref[...]
    for n in range(N):
        xn = x_ref[:, n, :]
        mu = jnp.mean(xn, axis=-1, keepdims=True)
        var = jnp.mean((xn - mu) ** 2, axis=-1, keepdims=True)
        x1 = (xn - mu) * jax.lax.rsqrt(var + 1e-5) * g + b
        qkv = jnp.dot(x1, wt, preferred_element_type=jnp.float32)
        for h in range(H):
            q_ref[:, h * N + n, :] = qkv[:, h * DH:(h + 1) * DH]
            k_ref[:, h * N + n, :] = qkv[:, C + h * DH:C + (h + 1) * DH]
        v_ref[:, n, :] = qkv[:, 2 * C:3 * C]
    q_ref[:, H * N, :] = jnp.zeros((q_ref.shape[0], DH), jnp.float32)
    k_ref[:, H * N, :] = jnp.zeros((k_ref.shape[0], DH), jnp.float32)


def _tc_qkv(x, g1, b1, wqkv_t):
    grid = (B // _BB,)
    return pl.pallas_call(
        _tc_qkv_body,
        grid=grid,
        in_specs=[
            pl.BlockSpec((_BB, N, C), lambda i: (i, 0, 0)),
            pl.BlockSpec((1, C), lambda i: (0, 0)),
            pl.BlockSpec((1, C), lambda i: (0, 0)),
            pl.BlockSpec((C, 3 * C), lambda i: (0, 0)),
        ],
        out_specs=[
            pl.BlockSpec((_BB, GR, DH), lambda i: (i, 0, 0)),
            pl.BlockSpec((_BB, GR, DH), lambda i: (i, 0, 0)),
            pl.BlockSpec((_BB, N, C), lambda i: (i, 0, 0)),
        ],
        out_shape=[
            jax.ShapeDtypeStruct((B, GR, DH), jnp.float32),
            jax.ShapeDtypeStruct((B, GR, DH), jnp.float32),
            jax.ShapeDtypeStruct((B, N, C), jnp.float32),
        ],
    )(x, g1, b1, wqkv_t)


_G_CH = 64
_EPW = E // NW


def _sc_gather_body(q_hbm, k_hbm, dst_hbm, src_hbm, qg_hbm, kg_hbm,
                    idx_v, buf, sem):
    wid = lax.axis_index("s") * NC + lax.axis_index("c")
    base = wid * _EPW
    for tab, idx_hbm, out in ((q_hbm, dst_hbm, qg_hbm),
                              (k_hbm, src_hbm, kg_hbm)):
        for c in range(_EPW // _G_CH):
            start = base + c * _G_CH
            pltpu.sync_copy(idx_hbm.at[pl.ds(start, _G_CH)], idx_v)
            pltpu.async_copy(tab.at[idx_v], buf, sem).wait()
            pltpu.sync_copy(buf, out.at[pl.ds(start, _G_CH)])


def _sc_gather(q2, k2, dst, src):
    mesh = plsc.VectorSubcoreMesh(core_axis_name="c", subcore_axis_name="s")
    fn = functools.partial(
        pl.kernel,
        mesh=mesh,
        out_type=[
            jax.ShapeDtypeStruct((E, GC), jnp.float32),
            jax.ShapeDtypeStruct((E, GC), jnp.float32),
        ],
        scratch_types=[
            pltpu.VMEM((_G_CH,), jnp.int32),
            pltpu.VMEM((_G_CH, GC), jnp.float32),
            pltpu.SemaphoreType.DMA,
        ],
    )(_sc_gather_body)
    return fn(q2, k2, dst, src)


def _tc_attn_body(q_ref, k_ref, r_ref):
    scale = DH ** -0.5
    parts = []
    for h in range(H):
        qh = q_ref[:, h * N:(h + 1) * N, :]
        kh = k_ref[:, h * N:(h + 1) * N, :]
        lg = jax.lax.dot_general(
            qh, kh, (((2,), (2,)), ((0,), (0,))),
            preferred_element_type=jnp.float32) * scale
        mx = jnp.max(lg, axis=-1, keepdims=True)
        ex = jnp.exp(lg - mx)
        sm = ex / jnp.sum(ex, axis=-1, keepdims=True)
        parts.append(sm.reshape(_EB, N * N))
    ones = jnp.ones((_EB, 1), jnp.float32)
    pad = jnp.zeros((_EB, R_COLS - A_COLS - 1), jnp.float32)
    r_ref[...] = jnp.concatenate(parts + [ones, pad], axis=-1)


def _tc_attn(qg3, kg3):
    grid = (E // _EB,)
    return pl.pallas_call(
        _tc_attn_body,
        grid=grid,
        in_specs=[
            pl.BlockSpec((_EB, GR, DH), lambda i: (i, 0, 0)),
            pl.BlockSpec((_EB, GR, DH), lambda i: (i, 0, 0)),
        ],
        out_specs=pl.BlockSpec((_EB, R_COLS), lambda i: (i, 0)),
        out_shape=jax.ShapeDtypeStruct((E, R_COLS), jnp.float32),
    )(qg3, kg3)


_EB2 = 512


def _tc_scatter_body(dst_ref, r_ref, s_ref, acc_ref):
    i = pl.program_id(0)

    @pl.when(i == 0)
    def _init():
        acc_ref[...] = jnp.zeros((B, R_COLS), jnp.float32)

    de = dst_ref[...]
    bi = jax.lax.broadcasted_iota(jnp.int32, (B, _EB2), 0)
    oh = jnp.where(bi == de, 1.0, 0.0)
    acc_ref[...] += jax.lax.dot_general(
        oh, r_ref[...], (((1,), (0,)), ((), ())),
        preferred_element_type=jnp.float32)

    @pl.when(i == pl.num_programs(0) - 1)
    def _done():
        s_ref[...] = acc_ref[...]


def _tc_scatter(r, dst):
    grid = (E // _EB2,)
    return pl.pallas_call(
        _tc_scatter_body,
        grid=grid,
        in_specs=[
            pl.BlockSpec((1, _EB2), lambda i: (0, i)),
            pl.BlockSpec((_EB2, R_COLS), lambda i: (i, 0)),
        ],
        out_specs=pl.BlockSpec((B, R_COLS), lambda i: (0, 0)),
        out_shape=jax.ShapeDtypeStruct((B, R_COLS), jnp.float32),
        scratch_shapes=[pltpu.VMEM((B, R_COLS), jnp.float32)],
    )(dst.reshape(1, E), r)


def _tc_out_body(s_ref, v_ref, x_ref, g2_ref, b2_ref, wp_ref, bp_ref,
                 wf1_ref, bf1_ref, wf2_ref, bf2_ref, o_ref):
    s = s_ref[...]
    cnt = s[:, A_COLS:A_COLS + 1]
    recip = 1.0 / jnp.maximum(cnt, 1.0)
    mask = (cnt > 0.0).astype(jnp.float32)
    heads = []
    for h in range(H):
        wh = (s[:, h * N * N:(h + 1) * N * N] * recip).reshape(_BB, N, N)
        vh = v_ref[:, :, h * DH:(h + 1) * DH]
        ah = jax.lax.dot_general(
            wh, vh, (((2,), (1,)), ((0,), (0,))),
            preferred_element_type=jnp.float32)
        heads.append(ah)
    a = jnp.concatenate(heads, axis=-1)

    g2 = g2_ref[...]
    b2 = b2_ref[...]
    wp = wp_ref[...]
    bp = bp_ref[...]
    wf1 = wf1_ref[...]
    bf1 = bf1_ref[...]
    wf2 = wf2_ref[...]
    bf2 = bf2_ref[...]
    inv_sqrt2 = 0.7071067811865476
    for n in range(N):
        yn = (jnp.dot(a[:, n, :], wp, preferred_element_type=jnp.float32)
              + bp) * mask
        x2n = x_ref[:, n, :] + yn
        mu = jnp.mean(x2n, axis=-1, keepdims=True)
        var = jnp.mean((x2n - mu) ** 2, axis=-1, keepdims=True)
        hn = (x2n - mu) * jax.lax.rsqrt(var + 1e-5) * g2 + b2
        f = jnp.dot(hn, wf1, preferred_element_type=jnp.float32) + bf1
        f = 0.5 * f * (1.0 + jax.lax.erf(f * inv_sqrt2))
        on = jnp.dot(f, wf2, preferred_element_type=jnp.float32) + bf2
        o_ref[:, n, :] = x2n + on


def _tc_out(s, v, x, g2, b2, wproj_t, bproj, wfc1_t, bfc1, wfc2_t, bfc2):
    grid = (B // _BB,)
    return pl.pallas_call(
        _tc_out_body,
        grid=grid,
        in_specs=[
            pl.BlockSpec((_BB, R_COLS), lambda i: (i, 0)),
            pl.BlockSpec((_BB, N, C), lambda i: (i, 0, 0)),
            pl.BlockSpec((_BB, N, C), lambda i: (i, 0, 0)),
            pl.BlockSpec((1, C), lambda i: (0, 0)),
            pl.BlockSpec((1, C), lambda i: (0, 0)),
            pl.BlockSpec((C, C), lambda i: (0, 0)),
            pl.BlockSpec((1, C), lambda i: (0, 0)),
            pl.BlockSpec((C, 4 * C), lambda i: (0, 0)),
            pl.BlockSpec((1, 4 * C), lambda i: (0, 0)),
            pl.BlockSpec((4 * C, C), lambda i: (0, 0)),
            pl.BlockSpec((1, C), lambda i: (0, 0)),
        ],
        out_specs=pl.BlockSpec((_BB, N, C), lambda i: (i, 0, 0)),
        out_shape=jax.ShapeDtypeStruct((B, N, C), jnp.float32),
    )(s, v, x, g2, b2, wproj_t, bproj, wfc1_t, bfc1, wfc2_t, bfc2)


def kernel(x, edge_index, g1, b1, Wqkv, Wproj, bproj, g2, b2,
           Wfc1, bfc1, Wfc2, bfc2):
    src = edge_index[0]
    dst = edge_index[1]

    q, k, v = _tc_qkv(x, g1.reshape(1, C), b1.reshape(1, C), Wqkv.T)

    qg, kg = _sc_gather(q.reshape(B, GC), k.reshape(B, GC), dst, src)

    r = _tc_attn(qg.reshape(E, GR, DH), kg.reshape(E, GR, DH))

    s = _tc_scatter(r, dst)

    out = _tc_out(s, v, x, g2.reshape(1, C), b2.reshape(1, C),
                  Wproj.T, bproj.reshape(1, C),
                  Wfc1.T, bfc1.reshape(1, 4 * C),
                  Wfc2.T, bfc2.reshape(1, C))
    return out
